# baseline (speedup 1.0000x reference)
import functools
import math

import jax
import jax.numpy as jnp
from jax import lax
from jax.experimental import pallas as pl
from jax.experimental.pallas import tpu as pltpu
from jax.experimental.pallas import tpu_sc as plsc

N = 10000
NP = 10240
E = 320000
Q = 320000
H = 128
D_EDGE = 16
NUM_LAYERS = 3
EPS = 1e-5

NC = 2
NS = 16
NW = NC * NS
RANGE = NP // NW
ACC_ROWS = RANGE + 1

SCAN_CHUNK = 2000
N_SCAN_CHUNKS = E // SCAN_CHUNK
STAGE = 2048
QCAP = E + 2 * STAGE

GBLK = 128
SB = STAGE // GBLK
CSRCAP = 49152

_SQRT2 = math.sqrt(2.0)


def _gelu(x):
    return 0.5 * x * (1.0 + lax.erf(x / _SQRT2))


_GDN = lax.GatherDimensionNumbers(
    offset_dims=(), collapsed_slice_dims=(0,), start_index_map=(0,))


def _lane_perm(x, idx):
    return lax.gather(x, idx[:, None], _GDN, (1,),
                      mode=lax.GatherScatterMode.PROMISE_IN_BOUNDS)


def _make_scan_kernel():
    mesh = plsc.VectorSubcoreMesh(core_axis_name="c", subcore_axis_name="s")

    @functools.partial(
        pl.kernel,
        mesh=mesh,
        out_type=(
            jax.ShapeDtypeStruct((NW * QCAP,), jnp.int32),
            jax.ShapeDtypeStruct((NW * QCAP,), jnp.int32),
            jax.ShapeDtypeStruct((NW * 16,), jnp.int32),
        ),
        compiler_params=pltpu.CompilerParams(needs_layout_passes=False),
        scratch_types=[
            pltpu.VMEM((SCAN_CHUNK,), jnp.int32),
            pltpu.VMEM((SCAN_CHUNK,), jnp.int32),
            pltpu.VMEM((STAGE + 16,), jnp.int32),
            pltpu.VMEM((STAGE + 16,), jnp.int32),
            pltpu.VMEM((16,), jnp.int32),
            pltpu.VMEM((336,), jnp.int32),
            pltpu.VMEM((336,), jnp.int32),
            pltpu.VMEM((CSRCAP,), jnp.int32),
            pltpu.VMEM((CSRCAP,), jnp.int32),
        ],
    )
    def scan_k(src_hbm, dst_hbm, qsrc_hbm, qdst_hbm, cnt_hbm,
               srcv, dstv, sstage, dstage, cntv, cnttab, cursors, csr_s, csr_d):
        wid = lax.axis_index("s") * NC + lax.axis_index("c")
        base = wid * RANGE
        hi = base + RANGE
        iota = lax.iota(jnp.int32, 16)
        sent_d = jnp.full((16,), 0, jnp.int32) + hi
        sent_s = jnp.zeros((16,), jnp.int32)

        def chunk_body(c, qcnt):
            pltpu.sync_copy(src_hbm.at[pl.ds(pl.multiple_of(c * SCAN_CHUNK, 8), SCAN_CHUNK)], srcv)
            pltpu.sync_copy(dst_hbm.at[pl.ds(pl.multiple_of(c * SCAN_CHUNK, 8), SCAN_CHUNK)], dstv)

            def vec_body(i, cnt):
                d = dstv[pl.ds(i * 16, 16)]
                s = srcv[pl.ds(i * 16, 16)]
                m = (d >= base) & (d < hi)
                mi = m.astype(jnp.int32)
                cs = plsc.cumsum(mi)
                tgt = cs - mi + cnt
                plsc.store_scatter(dstage, [tgt], d, mask=m)
                plsc.store_scatter(sstage, [tgt], s, mask=m)
                return cnt + jnp.max(cs)

            cnt = lax.fori_loop(0, SCAN_CHUNK // 16, vec_body, jnp.int32(0))
            dstage[pl.ds(cnt, 16)] = sent_d
            sstage[pl.ds(cnt, 16)] = sent_s
            pltpu.sync_copy(sstage.at[pl.ds(0, STAGE)],
                            qsrc_hbm.at[pl.ds(pl.multiple_of(wid * QCAP + qcnt, 8), STAGE)])
            pltpu.sync_copy(dstage.at[pl.ds(0, STAGE)],
                            qdst_hbm.at[pl.ds(pl.multiple_of(wid * QCAP + qcnt, 8), STAGE)])
            return qcnt + ((cnt + 7) & ~7)

        qcnt = lax.fori_loop(0, N_SCAN_CHUNKS, chunk_body, jnp.int32(0))

        def fill_body(i, _):
            dstage[pl.ds(i * 16, 16)] = sent_d
            sstage[pl.ds(i * 16, 16)] = sent_s
            return 0

        lax.fori_loop(0, (STAGE + 16) // 16, fill_body, 0)
        pltpu.sync_copy(sstage.at[pl.ds(0, STAGE)],
                        qsrc_hbm.at[pl.ds(pl.multiple_of(wid * QCAP + qcnt, 8), STAGE)])
        pltpu.sync_copy(dstage.at[pl.ds(0, STAGE)],
                        qdst_hbm.at[pl.ds(pl.multiple_of(wid * QCAP + qcnt, 8), STAGE)])
        nq0 = (qcnt + STAGE - 1) >> 11
        fast = qcnt <= CSRCAP - STAGE
        widq = wid * QCAP

        def run_boundaries(ks):
            prev = _lane_perm(ks, jnp.maximum(iota - 1, 0))
            head = (ks != prev) | (iota == 0)
            runstart = plsc.cummax(jnp.where(head, iota, 0))
            rank = iota - runstart
            nxt = _lane_perm(ks, jnp.minimum(iota + 1, 15))
            last = (ks != nxt) | (iota == 15)
            return rank, last

        def do_sort(_):
            def zt(i, _):
                cnttab[pl.ds(i * 16, 16)] = jnp.zeros((16,), jnp.int32)
                return 0

            lax.fori_loop(0, 336 // 16, zt, 0)

            def p2_chunk(qb, _):
                pltpu.sync_copy(
                    qdst_hbm.at[pl.ds(pl.multiple_of(widq + qb * STAGE, 8), STAGE)],
                    dstage.at[pl.ds(0, STAGE)])

                def p2_v(i, _):
                    lv = dstage[pl.ds(i * 16, 16)] - base
                    ks, _vs = plsc.sort_key_val(lv, iota)
                    rank, last = run_boundaries(ks)
                    cur = plsc.load_gather(cnttab, [ks])
                    plsc.store_scatter(cnttab, [ks], cur + rank + 1, mask=last)
                    return 0

                lax.fori_loop(0, STAGE // 16, p2_v, 0)
                return 0

            lax.fori_loop(0, nq0, p2_chunk, 0)

            def p3(k, c):
                v = cnttab[pl.ds(k * 16, 16)]
                cs = plsc.cumsum(v)
                cb = lax.broadcast_in_dim(c, (16,), ())
                cursors[pl.ds(k * 16, 16)] = cb + cs - v
                return c + cs[15]

            lax.fori_loop(0, 336 // 16, p3, jnp.int32(0))

            def p4_chunk(qb, _):
                pltpu.sync_copy(
                    qsrc_hbm.at[pl.ds(pl.multiple_of(widq + qb * STAGE, 8), STAGE)],
                    sstage.at[pl.ds(0, STAGE)])
                pltpu.sync_copy(
                    qdst_hbm.at[pl.ds(pl.multiple_of(widq + qb * STAGE, 8), STAGE)],
                    dstage.at[pl.ds(0, STAGE)])

                def p4_v(i, _):
                    d = dstage[pl.ds(i * 16, 16)]
                    s = sstage[pl.ds(i * 16, 16)]
                    lv = d - base
                    ks, perm = plsc.sort_key_val(lv, iota)
                    rank, last = run_boundaries(ks)
                    curs = plsc.load_gather(cursors, [ks])
                    tgt = curs + rank
                    plsc.store_scatter(csr_s, [tgt], _lane_perm(s, perm))
                    plsc.store_scatter(csr_d, [tgt], _lane_perm(d, perm))
                    plsc.store_scatter(cursors, [ks], curs + rank + 1, mask=last)
                    return 0

                lax.fori_loop(0, STAGE // 16, p4_v, 0)
                return 0

            lax.fori_loop(0, nq0, p4_chunk, 0)

            def p5(qb, _):
                pltpu.sync_copy(
                    csr_s.at[pl.ds(pl.multiple_of(qb * STAGE, 8), STAGE)],
                    qsrc_hbm.at[pl.ds(pl.multiple_of(widq + qb * STAGE, 8), STAGE)])
                pltpu.sync_copy(
                    csr_d.at[pl.ds(pl.multiple_of(qb * STAGE, 8), STAGE)],
                    qdst_hbm.at[pl.ds(pl.multiple_of(widq + qb * STAGE, 8), STAGE)])
                return 0

            lax.fori_loop(0, nq0, p5, 0)
            return 0

        lax.cond(fast, do_sort, lambda _: 0, 0)

        flag = fast.astype(jnp.int32)
        qsplat = lax.broadcast_in_dim(qcnt, (16,), ())
        fsplat = lax.broadcast_in_dim(flag, (16,), ())
        cntv[pl.ds(0, 16)] = jnp.where(iota == 1, fsplat, qsplat)
        pltpu.sync_copy(cntv, cnt_hbm.at[pl.ds(pl.multiple_of(wid * 16, 8), 16)])

    return scan_k


def _make_agg_kernel():
    mesh = plsc.VectorSubcoreMesh(core_axis_name="c", subcore_axis_name="s")

    @functools.partial(
        pl.kernel,
        mesh=mesh,
        out_type=(
            jax.ShapeDtypeStruct((NP * H,), jnp.float32),
            jax.ShapeDtypeStruct((NP * H,), jnp.float32),
        ),
        compiler_params=pltpu.CompilerParams(needs_layout_passes=False),
        scratch_types=[
            pltpu.VMEM((ACC_ROWS * H,), jnp.float32),
            pltpu.VMEM((ACC_ROWS * H,), jnp.float32),
            pltpu.VMEM((ACC_ROWS,), jnp.float32),
            pltpu.VMEM((2 * STAGE,), jnp.int32),
            pltpu.VMEM((2 * STAGE,), jnp.int32),
            pltpu.VMEM((GBLK,), jnp.int32),
            pltpu.VMEM((GBLK,), jnp.int32),
            pltpu.VMEM((GBLK,), jnp.int32),
            pltpu.VMEM((GBLK, H), jnp.float32),
            pltpu.VMEM((GBLK, H), jnp.float32),
            pltpu.VMEM((ACC_ROWS * 16,), jnp.float32),
            pltpu.VMEM((16,), jnp.int32),
            pltpu.SemaphoreType.DMA,
            pltpu.SemaphoreType.DMA,
        ],
    )
    def agg_k(h_hbm, qsrc_hbm, qdst_hbm, cnt_hbm, mean_hbm, max_hbm,
              accsum, accmax, accdeg, idxsb, dstsb, idxa, idxb, dstv,
              rva, rvb, cntarr, cntv, sema, semb):
        wid = lax.axis_index("s") * NC + lax.axis_index("c")
        base = wid * RANGE
        widq = wid * QCAP
        iota = lax.iota(jnp.int32, 16)
        cf = [jnp.full((16,), f * 16, jnp.int32) + iota for f in range(8)]
        zero16 = jnp.zeros((16,), jnp.float32)
        ninf16 = jnp.full((16,), -jnp.inf, jnp.float32)
        one16 = jnp.ones((16,), jnp.float32)
        lane0 = iota == 0

        pltpu.sync_copy(cnt_hbm.at[pl.ds(pl.multiple_of(wid * 16, 8), 16)], cntv)
        c16 = cntv[pl.ds(0, 16)]
        qcnt = c16[0]
        flag = c16[1]

        def fast_path(_):
            def initz(i, _):
                accsum[pl.ds(i * 16, 16)] = zero16
                accmax[pl.ds(i * 16, 16)] = zero16
                return 0

            lax.fori_loop(0, ACC_ROWS * H // 16, initz, 0)

            def initc(i, _):
                cntarr[pl.ds(i * 16, 16)] = zero16
                return 0

            lax.fori_loop(0, ACC_ROWS, initc, 0)

            nq = jnp.maximum((qcnt + STAGE - 1) >> 11, 1)
            nblk = nq * SB

            def load_sb(sb):
                par = (sb & 1) * STAGE
                pltpu.sync_copy(
                    qsrc_hbm.at[pl.ds(pl.multiple_of(widq + sb * STAGE, 8), STAGE)],
                    idxsb.at[pl.ds(pl.multiple_of(par, 8), STAGE)])
                pltpu.sync_copy(
                    qdst_hbm.at[pl.ds(pl.multiple_of(widq + sb * STAGE, 8), STAGE)],
                    dstsb.at[pl.ds(pl.multiple_of(par, 8), STAGE)])

            def fire(x, ibuf, rbuf, sem):
                off = ((x >> 4) & 1) * STAGE + (x & 15) * GBLK
                for k in range(8):
                    ibuf[pl.ds(k * 16, 16)] = idxsb[pl.ds(off + k * 16, 16)]
                pltpu.async_copy(h_hbm.at[ibuf], rbuf, sem)

            load_sb(jnp.int32(0))

            def process(b, rvi, C):
                boff = ((b >> 4) & 1) * STAGE + (b & 15) * GBLK

                def vbody(v, C):
                    dprev, cnt, sums, maxs = C
                    dvec = dstsb[pl.ds(boff + v * 16, 16)]
                    for j in range(16):
                        d_j = dvec[j]
                        evec = jnp.full((16,), 0, jnp.int32) + (v * 16 + j)
                        rows = [plsc.load_gather(rvi, [evec, cf[f]])
                                for f in range(8)]
                        same = d_j == dprev
                        cnt = jnp.where(same, cnt + 1, jnp.int32(1))
                        sums = [jnp.where(same, sums[f] + rows[f], rows[f])
                                for f in range(8)]
                        maxs = [jnp.where(same, jnp.maximum(maxs[f], rows[f]),
                                          rows[f]) for f in range(8)]
                        offv = lax.broadcast_in_dim((d_j - base) * H, (16,), ())
                        for f in range(8):
                            plsc.store_scatter(accsum, [offv + cf[f]], sums[f])
                            plsc.store_scatter(accmax, [offv + cf[f]], maxs[f])
                        cb = lax.broadcast_in_dim(cnt, (16,), ())
                        cvec = lax.broadcast_in_dim((d_j - base) * 16, (16,), ())
                        plsc.store_scatter(cntarr, [cvec + iota],
                                           cb.astype(jnp.float32))
                        dprev = d_j
                    return (dprev, cnt, sums, maxs)

                return lax.fori_loop(0, 8, vbody, C)

            C0 = (jnp.int32(-1), jnp.int32(0), [zero16] * 8, [zero16] * 8)

            def bb_body(b, C):
                def presb(_):
                    load_sb((b >> 4) + 1)
                    return 0

                lax.cond(((b & 15) == 0) & (b + 16 < nblk), presb,
                         lambda _: 0, 0)

                pltpu.sync_copy(
                    qsrc_hbm.at[pl.ds(pl.multiple_of(widq + b * GBLK, 8), GBLK)],
                    idxa)
                pltpu.async_copy(h_hbm.at[idxa], rva, sema).wait()
                C = process(b, rva, C)
                return C

            lax.fori_loop(0, nblk, bb_body, C0)

            def fin_fast(n, _):
                cv = cntarr[pl.ds(n * 16, 16)]
                inv = 1.0 / jnp.maximum(cv, 1.0)
                for f in range(8):
                    off = n * H + f * 16
                    accsum[pl.ds(off, 16)] = accsum[pl.ds(off, 16)] * inv
                return 0

            lax.fori_loop(0, RANGE, fin_fast, 0)
            return 0

        def slow_path(_):
            def init_body(i, _):
                accsum[pl.ds(i * 16, 16)] = zero16
                accmax[pl.ds(i * 16, 16)] = ninf16
                return 0

            lax.fori_loop(0, ACC_ROWS * H // 16, init_body, 0)

            def initd_body(i, _):
                accdeg[pl.ds(i * 16, 16)] = zero16
                return 0

            lax.fori_loop(0, (ACC_ROWS + 15) // 16, initd_body, 0)

            nblk = (qcnt + GBLK - 1) >> 7

            def blk_body(b, _):
                pltpu.sync_copy(
                    qsrc_hbm.at[pl.ds(pl.multiple_of(widq + b * GBLK, 8), GBLK)],
                    idxa)
                pltpu.sync_copy(
                    qdst_hbm.at[pl.ds(pl.multiple_of(widq + b * GBLK, 8), GBLK)],
                    dstv)
                pltpu.async_copy(h_hbm.at[idxa], rva, sema).wait()

                def edge_body(e, _):
                    evec = jnp.full((16,), 0, jnp.int32) + e
                    dvec = plsc.load_gather(dstv, [evec])
                    lvec = dvec - base
                    lbase = lvec * H
                    plsc.addupdate_scatter(accdeg, [lvec], one16, mask=lane0)
                    for f in range(8):
                        msg = plsc.load_gather(rva, [evec, cf[f]])
                        aidx = lbase + cf[f]
                        plsc.addupdate_scatter(accsum, [aidx], msg)
                        curm = plsc.load_gather(accmax, [aidx])
                        plsc.store_scatter(accmax, [aidx], jnp.maximum(curm, msg))
                    return 0

                lax.fori_loop(0, GBLK, edge_body, 0)
                return 0

            lax.fori_loop(0, nblk, blk_body, 0)

            def fin_body(n, _):
                nvec = jnp.full((16,), 0, jnp.int32) + n
                dsplat = plsc.load_gather(accdeg, [nvec])
                inv = 1.0 / jnp.maximum(dsplat, 1.0)
                nonempty = dsplat > 0.0
                for f in range(8):
                    off = n * H + f * 16
                    accsum[pl.ds(off, 16)] = accsum[pl.ds(off, 16)] * inv
                    mx = accmax[pl.ds(off, 16)]
                    accmax[pl.ds(off, 16)] = jnp.where(nonempty, mx, 0.0)
                return 0

            lax.fori_loop(0, RANGE, fin_body, 0)
            return 0

        lax.cond(flag == 1, fast_path, slow_path, 0)

        pltpu.sync_copy(accsum.at[pl.ds(0, RANGE * H)],
                        mean_hbm.at[pl.ds(pl.multiple_of(base * H, 8), RANGE * H)])
        pltpu.sync_copy(accmax.at[pl.ds(0, RANGE * H)],
                        max_hbm.at[pl.ds(pl.multiple_of(base * H, 8), RANGE * H)])

    return agg_k


def _make_qgather_kernel():
    mesh = plsc.VectorSubcoreMesh(core_axis_name="c", subcore_axis_name="s")
    B_W = Q // NW
    CB = 200
    NCH = B_W // CB

    @functools.partial(
        pl.kernel,
        mesh=mesh,
        out_type=(
            jax.ShapeDtypeStruct((Q, H), jnp.float32),
            jax.ShapeDtypeStruct((Q, H), jnp.float32),
        ),
        compiler_params=pltpu.CompilerParams(needs_layout_passes=False),
        scratch_types=[
            pltpu.VMEM((CB,), jnp.int32),
            pltpu.VMEM((CB, H), jnp.float32),
            pltpu.VMEM((CB,), jnp.int32),
            pltpu.VMEM((CB, H), jnp.float32),
            pltpu.SemaphoreType.DMA,
            pltpu.SemaphoreType.DMA,
        ],
    )
    def qg_k(h_hbm, qs_hbm, qt_hbm, outs_hbm, outt_hbm,
             idxs, rows, idxt, rowt, sems, semt):
        wid = lax.axis_index("s") * NC + lax.axis_index("c")
        qbase = wid * B_W

        def blk(b, _):
            off = pl.multiple_of(qbase + b * CB, 8)
            pltpu.sync_copy(qs_hbm.at[pl.ds(off, CB)], idxs)
            pltpu.sync_copy(qt_hbm.at[pl.ds(off, CB)], idxt)
            cs = pltpu.async_copy(h_hbm.at[idxs], rows, sems)
            ct = pltpu.async_copy(h_hbm.at[idxt], rowt, semt)
            cs.wait()
            pltpu.sync_copy(rows, outs_hbm.at[pl.ds(off, CB)])
            ct.wait()
            pltpu.sync_copy(rowt, outt_hbm.at[pl.ds(off, CB)])
            return 0

        lax.fori_loop(0, NCH, blk, 0)

    return qg_k


_ROWS_BLK = 1280


def _enc_body(x_ref, w_ref, b_ref, o_ref):
    o_ref[...] = (
        jnp.dot(x_ref[...], w_ref[...], preferred_element_type=jnp.float32)
        + b_ref[...]
    )


def _encoder(x, w, b):
    return pl.pallas_call(
        _enc_body,
        grid=(NP // _ROWS_BLK,),
        in_specs=[
            pl.BlockSpec((_ROWS_BLK, H), lambda i: (i, 0)),
            pl.BlockSpec((H, H), lambda i: (0, 0)),
            pl.BlockSpec((1, H), lambda i: (0, 0)),
        ],
        out_specs=pl.BlockSpec((_ROWS_BLK, H), lambda i: (i, 0)),
        out_shape=jax.ShapeDtypeStruct((NP, H), jnp.float32),
    )(x, w, b)


def _layer_body(mean_ref, max_ref, h_ref, wla_ref, wlb_ref, wr_ref,
                b_ref, g_ref, bln_ref, o_ref):
    h = h_ref[...]
    z = (
        jnp.dot(mean_ref[...], wla_ref[...], preferred_element_type=jnp.float32)
        + jnp.dot(max_ref[...], wlb_ref[...], preferred_element_type=jnp.float32)
        + jnp.dot(h, wr_ref[...], preferred_element_type=jnp.float32)
        + b_ref[...]
    )
    mu = jnp.mean(z, axis=1, keepdims=True)
    var = jnp.mean((z - mu) ** 2, axis=1, keepdims=True)
    zn = (z - mu) * lax.rsqrt(var + EPS) * g_ref[...] + bln_ref[...]
    o_ref[...] = _gelu(zn) + h


def _layer_update(mean, mx, h, wla, wlb, wr, b, g, bln):
    return pl.pallas_call(
        _layer_body,
        grid=(NP // _ROWS_BLK,),
        in_specs=[
            pl.BlockSpec((_ROWS_BLK, H), lambda i: (i, 0)),
            pl.BlockSpec((_ROWS_BLK, H), lambda i: (i, 0)),
            pl.BlockSpec((_ROWS_BLK, H), lambda i: (i, 0)),
            pl.BlockSpec((H, H), lambda i: (0, 0)),
            pl.BlockSpec((H, H), lambda i: (0, 0)),
            pl.BlockSpec((H, H), lambda i: (0, 0)),
            pl.BlockSpec((1, H), lambda i: (0, 0)),
            pl.BlockSpec((1, H), lambda i: (0, 0)),
            pl.BlockSpec((1, H), lambda i: (0, 0)),
        ],
        out_specs=pl.BlockSpec((_ROWS_BLK, H), lambda i: (i, 0)),
        out_shape=jax.ShapeDtypeStruct((NP, H), jnp.float32),
    )(mean, mx, h, wla, wlb, wr, b, g, bln)


_Q_BLK = 2000


def _mlp_body(hs_ref, ht_ref, ea_ref, w1a_ref, w1b_ref, w1c_ref, b1_ref,
              w2_ref, b2_ref, w3_ref, b3_ref, o_ref):
    z = (
        jnp.dot(hs_ref[...], w1a_ref[...], preferred_element_type=jnp.float32)
        + jnp.dot(ht_ref[...], w1b_ref[...], preferred_element_type=jnp.float32)
        + jnp.dot(ea_ref[...], w1c_ref[...], preferred_element_type=jnp.float32)
        + b1_ref[...]
    )
    z = _gelu(z)
    z = _gelu(
        jnp.dot(z, w2_ref[...], preferred_element_type=jnp.float32) + b2_ref[...]
    )
    o_ref[...] = (
        jnp.dot(z, w3_ref[...], preferred_element_type=jnp.float32) + b3_ref[...]
    )


def _edge_mlp(hs, ht, ea, w1a, w1b, w1c, b1, w2, b2, w3, b3):
    return pl.pallas_call(
        _mlp_body,
        grid=(Q // _Q_BLK,),
        in_specs=[
            pl.BlockSpec((_Q_BLK, H), lambda i: (i, 0)),
            pl.BlockSpec((_Q_BLK, H), lambda i: (i, 0)),
            pl.BlockSpec((_Q_BLK, D_EDGE), lambda i: (i, 0)),
            pl.BlockSpec((H, 2 * H), lambda i: (0, 0)),
            pl.BlockSpec((H, 2 * H), lambda i: (0, 0)),
            pl.BlockSpec((D_EDGE, 2 * H), lambda i: (0, 0)),
            pl.BlockSpec((1, 2 * H), lambda i: (0, 0)),
            pl.BlockSpec((2 * H, H), lambda i: (0, 0)),
            pl.BlockSpec((1, H), lambda i: (0, 0)),
            pl.BlockSpec((H, 1), lambda i: (0, 0)),
            pl.BlockSpec((1, 1), lambda i: (0, 0)),
        ],
        out_specs=pl.BlockSpec((_Q_BLK, 1), lambda i: (i, 0)),
        out_shape=jax.ShapeDtypeStruct((Q, 1), jnp.float32),
    )(hs, ht, ea, w1a, w1b, w1c, b1, w2, b2, w3, b3)


def kernel(x, edge_index, edge_attr, query_edge_indices, params):
    src = edge_index[0]
    dst = edge_index[1]
    qs = query_edge_indices[0]
    qt = query_edge_indices[1]

    xp = jnp.pad(x, ((0, NP - N), (0, 0)))

    scan_k = _make_scan_kernel()
    qsrc, qdst, qcnt = scan_k(src, dst)

    h = _encoder(xp, params["W_enc"], params["b_enc"][None, :])

    agg_k = _make_agg_kernel()
    for i in range(NUM_LAYERS):
        mean_f, max_f = agg_k(h, qsrc, qdst, qcnt)
        mean = mean_f.reshape(NP, H)
        mx = max_f.reshape(NP, H)
        wl = params["W_l"][i]
        h = _layer_update(
            mean, mx, h,
            wl[:H], wl[H:], params["W_r"][i],
            params["b_l"][i][None, :],
            params["ln_g"][i][None, :], params["ln_b"][i][None, :],
        )

    qg_k = _make_qgather_kernel()
    hqs, hqt = qg_k(h, qs, qt)

    bn_scale = params["bn_g"] / math.sqrt(1.0 + EPS)
    w1 = params["W1"] * bn_scale[None, :]
    b1 = params["b1"] * bn_scale + params["bn_b"]

    out = _edge_mlp(
        hqs, hqt, edge_attr,
        w1[:H], w1[H : 2 * H], w1[2 * H :], b1[None, :],
        params["W2"], params["b2"][None, :],
        params["W3"], params["b3"][None, :],
    )
    return out


# speedup vs baseline: 2.0031x; 2.0031x over previous
import functools
import math

import jax
import jax.numpy as jnp
from jax import lax
from jax.experimental import pallas as pl
from jax.experimental.pallas import tpu as pltpu
from jax.experimental.pallas import tpu_sc as plsc

N = 10000
NP = 10240
E = 320000
Q = 320000
H = 128
D_EDGE = 16
NUM_LAYERS = 3
EPS = 1e-5

NC = 2
NS = 16
NW = NC * NS
RANGE = NP // NW
ACC_ROWS = RANGE + 1

SCAN_CHUNK = 2000
N_SCAN_CHUNKS = E // SCAN_CHUNK
STAGE = 2048
QCAP = E + 2 * STAGE

GBLK = 128
SB = STAGE // GBLK
CSRCAP = 49152

_SQRT2 = math.sqrt(2.0)


def _gelu(x):
    return 0.5 * x * (1.0 + lax.erf(x / _SQRT2))


_GDN = lax.GatherDimensionNumbers(
    offset_dims=(), collapsed_slice_dims=(0,), start_index_map=(0,))


def _lane_perm(x, idx):
    return lax.gather(x, idx[:, None], _GDN, (1,),
                      mode=lax.GatherScatterMode.PROMISE_IN_BOUNDS)


def _make_scan_kernel():
    mesh = plsc.VectorSubcoreMesh(core_axis_name="c", subcore_axis_name="s")

    @functools.partial(
        pl.kernel,
        mesh=mesh,
        out_type=(
            jax.ShapeDtypeStruct((NW * QCAP,), jnp.int32),
            jax.ShapeDtypeStruct((NW * QCAP,), jnp.int32),
            jax.ShapeDtypeStruct((NW * 16,), jnp.int32),
        ),
        compiler_params=pltpu.CompilerParams(needs_layout_passes=False),
        scratch_types=[
            pltpu.VMEM((SCAN_CHUNK,), jnp.int32),
            pltpu.VMEM((SCAN_CHUNK,), jnp.int32),
            pltpu.VMEM((STAGE + 16,), jnp.int32),
            pltpu.VMEM((STAGE + 16,), jnp.int32),
            pltpu.VMEM((16,), jnp.int32),
            pltpu.VMEM((336,), jnp.int32),
            pltpu.VMEM((336,), jnp.int32),
            pltpu.VMEM((CSRCAP,), jnp.int32),
            pltpu.VMEM((CSRCAP,), jnp.int32),
        ],
    )
    def scan_k(src_hbm, dst_hbm, qsrc_hbm, qdst_hbm, cnt_hbm,
               srcv, dstv, sstage, dstage, cntv, cnttab, cursors, csr_s, csr_d):
        wid = lax.axis_index("s") * NC + lax.axis_index("c")
        base = wid * RANGE
        hi = base + RANGE
        iota = lax.iota(jnp.int32, 16)
        sent_d = jnp.full((16,), 0, jnp.int32) + hi
        sent_s = jnp.zeros((16,), jnp.int32)

        def chunk_body(c, qcnt):
            pltpu.sync_copy(src_hbm.at[pl.ds(pl.multiple_of(c * SCAN_CHUNK, 8), SCAN_CHUNK)], srcv)
            pltpu.sync_copy(dst_hbm.at[pl.ds(pl.multiple_of(c * SCAN_CHUNK, 8), SCAN_CHUNK)], dstv)

            def vec_body(i, cnt):
                d = dstv[pl.ds(i * 16, 16)]
                s = srcv[pl.ds(i * 16, 16)]
                m = (d >= base) & (d < hi)
                mi = m.astype(jnp.int32)
                cs = plsc.cumsum(mi)
                tgt = cs - mi + cnt
                plsc.store_scatter(dstage, [tgt], d, mask=m)
                plsc.store_scatter(sstage, [tgt], s, mask=m)
                return cnt + jnp.max(cs)

            cnt = lax.fori_loop(0, SCAN_CHUNK // 16, vec_body, jnp.int32(0))
            dstage[pl.ds(cnt, 16)] = sent_d
            sstage[pl.ds(cnt, 16)] = sent_s
            pltpu.sync_copy(sstage.at[pl.ds(0, STAGE)],
                            qsrc_hbm.at[pl.ds(pl.multiple_of(wid * QCAP + qcnt, 8), STAGE)])
            pltpu.sync_copy(dstage.at[pl.ds(0, STAGE)],
                            qdst_hbm.at[pl.ds(pl.multiple_of(wid * QCAP + qcnt, 8), STAGE)])
            return qcnt + ((cnt + 7) & ~7)

        qcnt = lax.fori_loop(0, N_SCAN_CHUNKS, chunk_body, jnp.int32(0))

        def fill_body(i, _):
            dstage[pl.ds(i * 16, 16)] = sent_d
            sstage[pl.ds(i * 16, 16)] = sent_s
            return 0

        lax.fori_loop(0, (STAGE + 16) // 16, fill_body, 0)
        pltpu.sync_copy(sstage.at[pl.ds(0, STAGE)],
                        qsrc_hbm.at[pl.ds(pl.multiple_of(wid * QCAP + qcnt, 8), STAGE)])
        pltpu.sync_copy(dstage.at[pl.ds(0, STAGE)],
                        qdst_hbm.at[pl.ds(pl.multiple_of(wid * QCAP + qcnt, 8), STAGE)])
        nq0 = (qcnt + STAGE - 1) >> 11
        fast = qcnt < 0
        widq = wid * QCAP

        def run_boundaries(ks):
            prev = _lane_perm(ks, jnp.maximum(iota - 1, 0))
            head = (ks != prev) | (iota == 0)
            runstart = plsc.cummax(jnp.where(head, iota, 0))
            rank = iota - runstart
            nxt = _lane_perm(ks, jnp.minimum(iota + 1, 15))
            last = (ks != nxt) | (iota == 15)
            return rank, last

        def do_sort(_):
            def zt(i, _):
                cnttab[pl.ds(i * 16, 16)] = jnp.zeros((16,), jnp.int32)
                return 0

            lax.fori_loop(0, 336 // 16, zt, 0)

            def p2_chunk(qb, _):
                pltpu.sync_copy(
                    qdst_hbm.at[pl.ds(pl.multiple_of(widq + qb * STAGE, 8), STAGE)],
                    dstage.at[pl.ds(0, STAGE)])

                def p2_v(i, _):
                    lv = dstage[pl.ds(i * 16, 16)] - base
                    ks, _vs = plsc.sort_key_val(lv, iota)
                    rank, last = run_boundaries(ks)
                    cur = plsc.load_gather(cnttab, [ks])
                    plsc.store_scatter(cnttab, [ks], cur + rank + 1, mask=last)
                    return 0

                lax.fori_loop(0, STAGE // 16, p2_v, 0)
                return 0

            lax.fori_loop(0, nq0, p2_chunk, 0)

            def p3(k, c):
                v = cnttab[pl.ds(k * 16, 16)]
                cs = plsc.cumsum(v)
                cb = lax.broadcast_in_dim(c, (16,), ())
                cursors[pl.ds(k * 16, 16)] = cb + cs - v
                return c + cs[15]

            lax.fori_loop(0, 336 // 16, p3, jnp.int32(0))

            def p4_chunk(qb, _):
                pltpu.sync_copy(
                    qsrc_hbm.at[pl.ds(pl.multiple_of(widq + qb * STAGE, 8), STAGE)],
                    sstage.at[pl.ds(0, STAGE)])
                pltpu.sync_copy(
                    qdst_hbm.at[pl.ds(pl.multiple_of(widq + qb * STAGE, 8), STAGE)],
                    dstage.at[pl.ds(0, STAGE)])

                def p4_v(i, _):
                    d = dstage[pl.ds(i * 16, 16)]
                    s = sstage[pl.ds(i * 16, 16)]
                    lv = d - base
                    ks, perm = plsc.sort_key_val(lv, iota)
                    rank, last = run_boundaries(ks)
                    curs = plsc.load_gather(cursors, [ks])
                    tgt = curs + rank
                    plsc.store_scatter(csr_s, [tgt], _lane_perm(s, perm))
                    plsc.store_scatter(csr_d, [tgt], _lane_perm(d, perm))
                    plsc.store_scatter(cursors, [ks], curs + rank + 1, mask=last)
                    return 0

                lax.fori_loop(0, STAGE // 16, p4_v, 0)
                return 0

            lax.fori_loop(0, nq0, p4_chunk, 0)

            def p5(qb, _):
                pltpu.sync_copy(
                    csr_s.at[pl.ds(pl.multiple_of(qb * STAGE, 8), STAGE)],
                    qsrc_hbm.at[pl.ds(pl.multiple_of(widq + qb * STAGE, 8), STAGE)])
                pltpu.sync_copy(
                    csr_d.at[pl.ds(pl.multiple_of(qb * STAGE, 8), STAGE)],
                    qdst_hbm.at[pl.ds(pl.multiple_of(widq + qb * STAGE, 8), STAGE)])
                return 0

            lax.fori_loop(0, nq0, p5, 0)
            return 0

        lax.cond(fast, do_sort, lambda _: 0, 0)

        flag = fast.astype(jnp.int32)
        qsplat = lax.broadcast_in_dim(qcnt, (16,), ())
        fsplat = lax.broadcast_in_dim(flag, (16,), ())
        cntv[pl.ds(0, 16)] = jnp.where(iota == 1, fsplat, qsplat)
        pltpu.sync_copy(cntv, cnt_hbm.at[pl.ds(pl.multiple_of(wid * 16, 8), 16)])

    return scan_k


def _make_agg_kernel():
    mesh = plsc.VectorSubcoreMesh(core_axis_name="c", subcore_axis_name="s")

    @functools.partial(
        pl.kernel,
        mesh=mesh,
        out_type=(
            jax.ShapeDtypeStruct((NP * H,), jnp.float32),
            jax.ShapeDtypeStruct((NP * H,), jnp.float32),
        ),
        compiler_params=pltpu.CompilerParams(needs_layout_passes=False),
        scratch_types=[
            pltpu.VMEM((ACC_ROWS * H,), jnp.float32),
            pltpu.VMEM((ACC_ROWS * H,), jnp.float32),
            pltpu.VMEM((ACC_ROWS,), jnp.float32),
            pltpu.VMEM((2 * STAGE,), jnp.int32),
            pltpu.VMEM((2 * STAGE,), jnp.int32),
            pltpu.VMEM((GBLK,), jnp.int32),
            pltpu.VMEM((GBLK,), jnp.int32),
            pltpu.VMEM((GBLK,), jnp.int32),
            pltpu.VMEM((GBLK, H), jnp.float32),
            pltpu.VMEM((GBLK, H), jnp.float32),
            pltpu.VMEM((ACC_ROWS * 16,), jnp.float32),
            pltpu.VMEM((16,), jnp.int32),
            pltpu.SemaphoreType.DMA,
            pltpu.SemaphoreType.DMA,
        ],
    )
    def agg_k(h_hbm, qsrc_hbm, qdst_hbm, cnt_hbm, mean_hbm, max_hbm,
              accsum, accmax, accdeg, idxsb, dstsb, idxa, idxb, dstv,
              rva, rvb, cntarr, cntv, sema, semb):
        wid = lax.axis_index("s") * NC + lax.axis_index("c")
        base = wid * RANGE
        widq = wid * QCAP
        iota = lax.iota(jnp.int32, 16)
        cf = [jnp.full((16,), f * 16, jnp.int32) + iota for f in range(8)]
        zero16 = jnp.zeros((16,), jnp.float32)
        ninf16 = jnp.full((16,), -jnp.inf, jnp.float32)
        one16 = jnp.ones((16,), jnp.float32)
        lane0 = iota == 0

        pltpu.sync_copy(cnt_hbm.at[pl.ds(pl.multiple_of(wid * 16, 8), 16)], cntv)
        c16 = cntv[pl.ds(0, 16)]
        qcnt = c16[0]
        flag = c16[1]

        def fast_path(_):
            def initz(i, _):
                accsum[pl.ds(i * 16, 16)] = zero16
                accmax[pl.ds(i * 16, 16)] = zero16
                return 0

            lax.fori_loop(0, ACC_ROWS * H // 16, initz, 0)

            def initc(i, _):
                cntarr[pl.ds(i * 16, 16)] = zero16
                return 0

            lax.fori_loop(0, ACC_ROWS, initc, 0)

            nq = jnp.maximum((qcnt + STAGE - 1) >> 11, 1)
            nblk = nq * SB

            def load_sb(sb):
                par = (sb & 1) * STAGE
                pltpu.sync_copy(
                    qsrc_hbm.at[pl.ds(pl.multiple_of(widq + sb * STAGE, 8), STAGE)],
                    idxsb.at[pl.ds(pl.multiple_of(par, 8), STAGE)])
                pltpu.sync_copy(
                    qdst_hbm.at[pl.ds(pl.multiple_of(widq + sb * STAGE, 8), STAGE)],
                    dstsb.at[pl.ds(pl.multiple_of(par, 8), STAGE)])

            def fire(x, ibuf, rbuf, sem):
                off = ((x >> 4) & 1) * STAGE + (x & 15) * GBLK
                for k in range(8):
                    ibuf[pl.ds(k * 16, 16)] = idxsb[pl.ds(off + k * 16, 16)]
                pltpu.async_copy(h_hbm.at[ibuf], rbuf, sem)

            load_sb(jnp.int32(0))

            def process(b, rvi, C):
                boff = ((b >> 4) & 1) * STAGE + (b & 15) * GBLK

                def vbody(v, C):
                    dprev, cnt, sums, maxs = C
                    dvec = dstsb[pl.ds(boff + v * 16, 16)]
                    for j in range(16):
                        d_j = dvec[j]
                        evec = jnp.full((16,), 0, jnp.int32) + (v * 16 + j)
                        rows = [plsc.load_gather(rvi, [evec, cf[f]])
                                for f in range(8)]
                        same = d_j == dprev
                        cnt = jnp.where(same, cnt + 1, jnp.int32(1))
                        sums = [jnp.where(same, sums[f] + rows[f], rows[f])
                                for f in range(8)]
                        maxs = [jnp.where(same, jnp.maximum(maxs[f], rows[f]),
                                          rows[f]) for f in range(8)]
                        offv = lax.broadcast_in_dim((d_j - base) * H, (16,), ())
                        for f in range(8):
                            plsc.store_scatter(accsum, [offv + cf[f]], sums[f])
                            plsc.store_scatter(accmax, [offv + cf[f]], maxs[f])
                        cb = lax.broadcast_in_dim(cnt, (16,), ())
                        cvec = lax.broadcast_in_dim((d_j - base) * 16, (16,), ())
                        plsc.store_scatter(cntarr, [cvec + iota],
                                           cb.astype(jnp.float32))
                        dprev = d_j
                    return (dprev, cnt, sums, maxs)

                return lax.fori_loop(0, 8, vbody, C)

            C0 = (jnp.int32(-1), jnp.int32(0), [zero16] * 8, [zero16] * 8)

            def bb_body(b, C):
                def presb(_):
                    load_sb((b >> 4) + 1)
                    return 0

                lax.cond(((b & 15) == 0) & (b + 16 < nblk), presb,
                         lambda _: 0, 0)

                pltpu.sync_copy(
                    qsrc_hbm.at[pl.ds(pl.multiple_of(widq + b * GBLK, 8), GBLK)],
                    idxa)
                pltpu.async_copy(h_hbm.at[idxa], rva, sema).wait()
                C = process(b, rva, C)
                return C

            lax.fori_loop(0, nblk, bb_body, C0)

            def fin_fast(n, _):
                cv = cntarr[pl.ds(n * 16, 16)]
                inv = 1.0 / jnp.maximum(cv, 1.0)
                for f in range(8):
                    off = n * H + f * 16
                    accsum[pl.ds(off, 16)] = accsum[pl.ds(off, 16)] * inv
                return 0

            lax.fori_loop(0, RANGE, fin_fast, 0)
            return 0

        def slow_path(_):
            def init_body(i, _):
                accsum[pl.ds(i * 16, 16)] = zero16
                accmax[pl.ds(i * 16, 16)] = ninf16
                return 0

            lax.fori_loop(0, ACC_ROWS * H // 16, init_body, 0)

            def initd_body(i, _):
                accdeg[pl.ds(i * 16, 16)] = zero16
                return 0

            lax.fori_loop(0, (ACC_ROWS + 15) // 16, initd_body, 0)

            nblk = (qcnt + GBLK - 1) >> 7

            def blk_body(b, _):
                pltpu.sync_copy(
                    qsrc_hbm.at[pl.ds(pl.multiple_of(widq + b * GBLK, 8), GBLK)],
                    idxa)
                pltpu.sync_copy(
                    qdst_hbm.at[pl.ds(pl.multiple_of(widq + b * GBLK, 8), GBLK)],
                    dstv)
                pltpu.async_copy(h_hbm.at[idxa], rva, sema).wait()

                def edge_body(e, _):
                    evec = jnp.full((16,), 0, jnp.int32) + e
                    dvec = plsc.load_gather(dstv, [evec])
                    lvec = dvec - base
                    lbase = lvec * H
                    plsc.addupdate_scatter(accdeg, [lvec], one16, mask=lane0)
                    for f in range(8):
                        msg = plsc.load_gather(rva, [evec, cf[f]])
                        aidx = lbase + cf[f]
                        plsc.addupdate_scatter(accsum, [aidx], msg)
                        curm = plsc.load_gather(accmax, [aidx])
                        plsc.store_scatter(accmax, [aidx], jnp.maximum(curm, msg))
                    return 0

                lax.fori_loop(0, GBLK, edge_body, 0)
                return 0

            lax.fori_loop(0, nblk, blk_body, 0)

            def fin_body(n, _):
                nvec = jnp.full((16,), 0, jnp.int32) + n
                dsplat = plsc.load_gather(accdeg, [nvec])
                inv = 1.0 / jnp.maximum(dsplat, 1.0)
                nonempty = dsplat > 0.0
                for f in range(8):
                    off = n * H + f * 16
                    accsum[pl.ds(off, 16)] = accsum[pl.ds(off, 16)] * inv
                    mx = accmax[pl.ds(off, 16)]
                    accmax[pl.ds(off, 16)] = jnp.where(nonempty, mx, 0.0)
                return 0

            lax.fori_loop(0, RANGE, fin_body, 0)
            return 0

        lax.cond(flag == 1, fast_path, slow_path, 0)

        pltpu.sync_copy(accsum.at[pl.ds(0, RANGE * H)],
                        mean_hbm.at[pl.ds(pl.multiple_of(base * H, 8), RANGE * H)])
        pltpu.sync_copy(accmax.at[pl.ds(0, RANGE * H)],
                        max_hbm.at[pl.ds(pl.multiple_of(base * H, 8), RANGE * H)])

    return agg_k


def _make_qgather_kernel():
    mesh = plsc.VectorSubcoreMesh(core_axis_name="c", subcore_axis_name="s")
    B_W = Q // NW
    CB = 200
    NCH = B_W // CB

    @functools.partial(
        pl.kernel,
        mesh=mesh,
        out_type=(
            jax.ShapeDtypeStruct((Q, H), jnp.float32),
            jax.ShapeDtypeStruct((Q, H), jnp.float32),
        ),
        compiler_params=pltpu.CompilerParams(needs_layout_passes=False),
        scratch_types=[
            pltpu.VMEM((CB,), jnp.int32),
            pltpu.VMEM((CB, H), jnp.float32),
            pltpu.VMEM((CB,), jnp.int32),
            pltpu.VMEM((CB, H), jnp.float32),
            pltpu.SemaphoreType.DMA,
            pltpu.SemaphoreType.DMA,
        ],
    )
    def qg_k(h_hbm, qs_hbm, qt_hbm, outs_hbm, outt_hbm,
             idxs, rows, idxt, rowt, sems, semt):
        wid = lax.axis_index("s") * NC + lax.axis_index("c")
        qbase = wid * B_W

        def blk(b, _):
            off = pl.multiple_of(qbase + b * CB, 8)
            pltpu.sync_copy(qs_hbm.at[pl.ds(off, CB)], idxs)
            pltpu.sync_copy(qt_hbm.at[pl.ds(off, CB)], idxt)
            cs = pltpu.async_copy(h_hbm.at[idxs], rows, sems)
            ct = pltpu.async_copy(h_hbm.at[idxt], rowt, semt)
            cs.wait()
            pltpu.sync_copy(rows, outs_hbm.at[pl.ds(off, CB)])
            ct.wait()
            pltpu.sync_copy(rowt, outt_hbm.at[pl.ds(off, CB)])
            return 0

        lax.fori_loop(0, NCH, blk, 0)

    return qg_k


_ROWS_BLK = 1280


def _enc_body(x_ref, w_ref, b_ref, o_ref):
    o_ref[...] = (
        jnp.dot(x_ref[...], w_ref[...], preferred_element_type=jnp.float32)
        + b_ref[...]
    )


def _encoder(x, w, b):
    return pl.pallas_call(
        _enc_body,
        grid=(NP // _ROWS_BLK,),
        in_specs=[
            pl.BlockSpec((_ROWS_BLK, H), lambda i: (i, 0)),
            pl.BlockSpec((H, H), lambda i: (0, 0)),
            pl.BlockSpec((1, H), lambda i: (0, 0)),
        ],
        out_specs=pl.BlockSpec((_ROWS_BLK, H), lambda i: (i, 0)),
        out_shape=jax.ShapeDtypeStruct((NP, H), jnp.float32),
    )(x, w, b)


def _layer_body(mean_ref, max_ref, h_ref, wla_ref, wlb_ref, wr_ref,
                b_ref, g_ref, bln_ref, o_ref):
    h = h_ref[...]
    z = (
        jnp.dot(mean_ref[...], wla_ref[...], preferred_element_type=jnp.float32)
        + jnp.dot(max_ref[...], wlb_ref[...], preferred_element_type=jnp.float32)
        + jnp.dot(h, wr_ref[...], preferred_element_type=jnp.float32)
        + b_ref[...]
    )
    mu = jnp.mean(z, axis=1, keepdims=True)
    var = jnp.mean((z - mu) ** 2, axis=1, keepdims=True)
    zn = (z - mu) * lax.rsqrt(var + EPS) * g_ref[...] + bln_ref[...]
    o_ref[...] = _gelu(zn) + h


def _layer_update(mean, mx, h, wla, wlb, wr, b, g, bln):
    return pl.pallas_call(
        _layer_body,
        grid=(NP // _ROWS_BLK,),
        in_specs=[
            pl.BlockSpec((_ROWS_BLK, H), lambda i: (i, 0)),
            pl.BlockSpec((_ROWS_BLK, H), lambda i: (i, 0)),
            pl.BlockSpec((_ROWS_BLK, H), lambda i: (i, 0)),
            pl.BlockSpec((H, H), lambda i: (0, 0)),
            pl.BlockSpec((H, H), lambda i: (0, 0)),
            pl.BlockSpec((H, H), lambda i: (0, 0)),
            pl.BlockSpec((1, H), lambda i: (0, 0)),
            pl.BlockSpec((1, H), lambda i: (0, 0)),
            pl.BlockSpec((1, H), lambda i: (0, 0)),
        ],
        out_specs=pl.BlockSpec((_ROWS_BLK, H), lambda i: (i, 0)),
        out_shape=jax.ShapeDtypeStruct((NP, H), jnp.float32),
    )(mean, mx, h, wla, wlb, wr, b, g, bln)


_Q_BLK = 2000


def _mlp_body(hs_ref, ht_ref, ea_ref, w1a_ref, w1b_ref, w1c_ref, b1_ref,
              w2_ref, b2_ref, w3_ref, b3_ref, o_ref):
    z = (
        jnp.dot(hs_ref[...], w1a_ref[...], preferred_element_type=jnp.float32)
        + jnp.dot(ht_ref[...], w1b_ref[...], preferred_element_type=jnp.float32)
        + jnp.dot(ea_ref[...], w1c_ref[...], preferred_element_type=jnp.float32)
        + b1_ref[...]
    )
    z = _gelu(z)
    z = _gelu(
        jnp.dot(z, w2_ref[...], preferred_element_type=jnp.float32) + b2_ref[...]
    )
    o_ref[...] = (
        jnp.dot(z, w3_ref[...], preferred_element_type=jnp.float32) + b3_ref[...]
    )


def _edge_mlp(hs, ht, ea, w1a, w1b, w1c, b1, w2, b2, w3, b3):
    return pl.pallas_call(
        _mlp_body,
        grid=(Q // _Q_BLK,),
        in_specs=[
            pl.BlockSpec((_Q_BLK, H), lambda i: (i, 0)),
            pl.BlockSpec((_Q_BLK, H), lambda i: (i, 0)),
            pl.BlockSpec((_Q_BLK, D_EDGE), lambda i: (i, 0)),
            pl.BlockSpec((H, 2 * H), lambda i: (0, 0)),
            pl.BlockSpec((H, 2 * H), lambda i: (0, 0)),
            pl.BlockSpec((D_EDGE, 2 * H), lambda i: (0, 0)),
            pl.BlockSpec((1, 2 * H), lambda i: (0, 0)),
            pl.BlockSpec((2 * H, H), lambda i: (0, 0)),
            pl.BlockSpec((1, H), lambda i: (0, 0)),
            pl.BlockSpec((H, 1), lambda i: (0, 0)),
            pl.BlockSpec((1, 1), lambda i: (0, 0)),
        ],
        out_specs=pl.BlockSpec((_Q_BLK, 1), lambda i: (i, 0)),
        out_shape=jax.ShapeDtypeStruct((Q, 1), jnp.float32),
    )(hs, ht, ea, w1a, w1b, w1c, b1, w2, b2, w3, b3)


def kernel(x, edge_index, edge_attr, query_edge_indices, params):
    src = edge_index[0]
    dst = edge_index[1]
    qs = query_edge_indices[0]
    qt = query_edge_indices[1]

    xp = jnp.pad(x, ((0, NP - N), (0, 0)))

    scan_k = _make_scan_kernel()
    qsrc, qdst, qcnt = scan_k(src, dst)

    h = _encoder(xp, params["W_enc"], params["b_enc"][None, :])

    agg_k = _make_agg_kernel()
    for i in range(NUM_LAYERS):
        mean_f, max_f = agg_k(h, qsrc, qdst, qcnt)
        mean = mean_f.reshape(NP, H)
        mx = max_f.reshape(NP, H)
        wl = params["W_l"][i]
        h = _layer_update(
            mean, mx, h,
            wl[:H], wl[H:], params["W_r"][i],
            params["b_l"][i][None, :],
            params["ln_g"][i][None, :], params["ln_b"][i][None, :],
        )

    qg_k = _make_qgather_kernel()
    hqs, hqt = qg_k(h, qs, qt)

    bn_scale = params["bn_g"] / math.sqrt(1.0 + EPS)
    w1 = params["W1"] * bn_scale[None, :]
    b1 = params["b1"] * bn_scale + params["bn_b"]

    out = _edge_mlp(
        hqs, hqt, edge_attr,
        w1[:H], w1[H : 2 * H], w1[2 * H :], b1[None, :],
        params["W2"], params["b2"][None, :],
        params["W3"], params["b3"][None, :],
    )
    return out
